# baseline (speedup 1.0000x reference)
import functools

import jax
import jax.numpy as jnp
from jax import lax
from jax.experimental import pallas as pl
from jax.experimental.pallas import tpu as pltpu
from jax.experimental.pallas import tpu_sc as plsc

_IDX_ROW = 128
_CH_ROWS = 16
_CHUNK = _IDX_ROW * _CH_ROWS


@functools.partial(jax.jit, static_argnames=("n_workers",))
def _embed_lookup(x2d, table, n_workers):
    n_idx_rows, _ = x2d.shape
    b_total = n_idx_rows * _IDX_ROW
    d = table.shape[1]
    rows_per_w = n_idx_rows // n_workers
    steps = rows_per_w // _CH_ROWS
    b_per_w = b_total // n_workers

    mesh = plsc.VectorSubcoreMesh(core_axis_name="c", subcore_axis_name="s")

    @functools.partial(
        pl.kernel,
        mesh=mesh,
        out_type=jax.ShapeDtypeStruct((b_total, d), jnp.float32),
        scratch_types=[
            pltpu.VMEM((_CH_ROWS, _IDX_ROW), jnp.int32),
            pltpu.VMEM((_CHUNK, d), jnp.float32),
            pltpu.SemaphoreType.DMA,
        ],
        compiler_params=pltpu.CompilerParams(use_tc_tiling_on_sc=False),
    )
    def k(table_hbm, idx_hbm, out_hbm, idx_v, rows_v, sem):
        wid = lax.axis_index("s") * 2 + lax.axis_index("c")
        ridx0 = wid * rows_per_w
        out0 = wid * b_per_w

        def step(s, carry):
            pltpu.sync_copy(
                idx_hbm.at[pl.ds(ridx0 + s * _CH_ROWS, _CH_ROWS)], idx_v)
            copies = [
                pltpu.async_copy(
                    table_hbm.at[idx_v.at[j]],
                    rows_v.at[pl.ds(j * _IDX_ROW, _IDX_ROW)],
                    sem)
                for j in range(_CH_ROWS)
            ]
            for cp in copies:
                cp.wait()
            pltpu.sync_copy(rows_v, out_hbm.at[pl.ds(out0 + s * _CHUNK, _CHUNK)])
            return carry

        lax.fori_loop(0, steps, step, 0)

    return k(table, x2d)


def kernel(x, table):
    b = x.shape[0]
    x2d = x.astype(jnp.int32).reshape(b // _IDX_ROW, _IDX_ROW)
    return _embed_lookup(x2d, table, 32)


# speedup vs baseline: 1.0210x; 1.0210x over previous
import functools

import jax
import jax.numpy as jnp
from jax import lax
from jax.experimental import pallas as pl
from jax.experimental.pallas import tpu as pltpu
from jax.experimental.pallas import tpu_sc as plsc

_IDX_ROW = 128
_CH_ROWS = 16
_CHUNK = _IDX_ROW * _CH_ROWS


@functools.partial(jax.jit, static_argnames=("n_workers",))
def _embed_lookup(x2d, table, n_workers):
    n_idx_rows, _ = x2d.shape
    b_total = n_idx_rows * _IDX_ROW
    d = table.shape[1]
    rows_per_w = n_idx_rows // n_workers
    steps = rows_per_w // _CH_ROWS
    b_per_w = b_total // n_workers

    mesh = plsc.VectorSubcoreMesh(core_axis_name="c", subcore_axis_name="s")

    assert steps % 2 == 0
    half = steps // 2

    @functools.partial(
        pl.kernel,
        mesh=mesh,
        out_type=jax.ShapeDtypeStruct((b_total, d), jnp.float32),
        scratch_types=[
            pltpu.VMEM((_CH_ROWS, _IDX_ROW), jnp.int32),
            pltpu.VMEM((_CH_ROWS, _IDX_ROW), jnp.int32),
            pltpu.VMEM((_CHUNK, d), jnp.float32),
            pltpu.VMEM((_CHUNK, d), jnp.float32),
            pltpu.SemaphoreType.DMA,
            pltpu.SemaphoreType.DMA,
            pltpu.SemaphoreType.DMA,
            pltpu.SemaphoreType.DMA,
        ],
        compiler_params=pltpu.CompilerParams(use_tc_tiling_on_sc=False),
    )
    def k(table_hbm, idx_hbm, out_hbm, idx0, idx1, buf0, buf1,
          sg0, sg1, so0, so1):
        wid = lax.axis_index("s") * 2 + lax.axis_index("c")
        ridx0 = wid * rows_per_w
        out0 = wid * b_per_w

        idx_bufs = (idx0, idx1)
        row_bufs = (buf0, buf1)
        g_sems = (sg0, sg1)
        o_sems = (so0, so1)

        def load_idx(s, par):
            pltpu.sync_copy(
                idx_hbm.at[pl.ds(ridx0 + s * _CH_ROWS, _CH_ROWS)],
                idx_bufs[par])

        def fire_gathers(par):
            return [
                pltpu.async_copy(
                    table_hbm.at[idx_bufs[par].at[j]],
                    row_bufs[par].at[pl.ds(j * _IDX_ROW, _IDX_ROW)],
                    g_sems[par])
                for j in range(_CH_ROWS)
            ]

        def gather_waits(par):
            for j in range(_CH_ROWS):
                pltpu.make_async_copy(
                    table_hbm.at[idx_bufs[par].at[j]],
                    row_bufs[par].at[pl.ds(j * _IDX_ROW, _IDX_ROW)],
                    g_sems[par]).wait()

        def fire_store(s, par):
            return pltpu.async_copy(
                row_bufs[par], out_hbm.at[pl.ds(out0 + s * _CHUNK, _CHUNK)],
                o_sems[par])

        def wait_store(s, par):
            pltpu.make_async_copy(
                row_bufs[par], out_hbm.at[pl.ds(out0 + s * _CHUNK, _CHUNK)],
                o_sems[par]).wait()

        load_idx(0, 0)
        fire_gathers(0)

        def body(g, carry):
            s0 = g * 2
            load_idx(s0 + 1, 1)
            gather_waits(0)
            fire_store(s0, 0)

            @pl.when(g > 0)
            def _():
                wait_store(s0 - 1, 1)

            fire_gathers(1)

            @pl.when(g < half - 1)
            def _():
                load_idx(s0 + 2, 0)
            gather_waits(1)
            fire_store(s0 + 1, 1)
            wait_store(s0, 0)

            @pl.when(g < half - 1)
            def _():
                fire_gathers(0)
            return carry

        lax.fori_loop(0, half, body, 0)
        wait_store(steps - 1, 1)

    return k(table, x2d)


def kernel(x, table):
    b = x.shape[0]
    x2d = x.astype(jnp.int32).reshape(b // _IDX_ROW, _IDX_ROW)
    return _embed_lookup(x2d, table, 32)
